# baseline (speedup 1.0000x reference)
import functools

import jax
import jax.numpy as jnp
from jax import lax
from jax.experimental import pallas as pl
from jax.experimental.pallas import tpu as pltpu
from jax.experimental.pallas import tpu_sc as plsc

N = 262144
NX = 848
NY = 848
NCELL = NX * NY
NWORKERS = 32
SLICE = 22496
NPAD = NWORKERS * SLICE
BMW = SLICE // 2
CHUNK = 1024
NCHUNK = N // CHUNK
RECW = 3 * CHUNK
UNROLL = 4
HB = 10
SENTINEL = 0xFFFFF

_F32_NEG_INF = float("-inf")


def _prep_body(x_ref, y_ref, z_ref, pk_ref, c0_ref, c1_ref, c2_ref):
    x = x_ref[...]
    y = y_ref[...]
    z = z_ref[...]
    cx = jnp.floor((x - jnp.float32(-84.8)) / jnp.float32(0.2)).astype(jnp.int32)
    cy = jnp.floor((y - jnp.float32(-84.8)) / jnp.float32(0.2)).astype(jnp.int32)
    cz = jnp.floor((z - jnp.float32(-3.0)) / jnp.float32(5.0)).astype(jnp.int32)
    valid = (
        (cx >= 0) & (cx < NX) & (cy >= 0) & (cy < NY) & (cz >= 0) & (cz < 1)
    )
    cell = cy * NX + cx
    hbin = jnp.clip(
        jnp.floor((z - jnp.float32(-3.0)) / jnp.float32(0.5)).astype(jnp.int32),
        0,
        HB - 1,
    )
    packed = jnp.where(valid, (hbin << 20) | cell, SENTINEL)
    pk_ref[...] = packed
    zero = jnp.zeros_like(cell)
    neg1 = zero - 1
    c0_ref[...] = jnp.where(valid, zero, neg1)
    c1_ref[...] = jnp.where(valid, cy, neg1)
    c2_ref[...] = jnp.where(valid, cx, neg1)


_prep_call = pl.pallas_call(
    _prep_body,
    out_shape=[
        jax.ShapeDtypeStruct((2048, 128), jnp.int32),
        jax.ShapeDtypeStruct((2048, 128), jnp.int32),
        jax.ShapeDtypeStruct((2048, 128), jnp.int32),
        jax.ShapeDtypeStruct((2048, 128), jnp.int32),
    ],
)


def _sc_body(
    rec_hbm,
    cnt_o,
    sz_o,
    si_o,
    mz_o,
    mi_o,
    bm_o,
    cnt_v,
    sz_v,
    si_v,
    mz_v,
    mi_v,
    bm_v,
    buf0,
    buf1,
    sem0,
    sem1,
):
    wid = lax.axis_index("s") * 2 + lax.axis_index("c")
    base = wid * SLICE

    zeros16 = jnp.zeros((16,), jnp.float32)
    ninf16 = jnp.full((16,), _F32_NEG_INF, jnp.float32)
    ones16 = jnp.ones((16,), jnp.float32)
    izeros16 = jnp.zeros((16,), jnp.int32)

    def init(j, _):
        off = j * 16
        cnt_v[pl.ds(off, 16)] = zeros16
        sz_v[pl.ds(off, 16)] = zeros16
        si_v[pl.ds(off, 16)] = zeros16
        mz_v[pl.ds(off, 16)] = ninf16
        mi_v[pl.ds(off, 16)] = ninf16
        return 0

    lax.fori_loop(0, SLICE // 16, init, 0)

    def init_bm(j, _):
        bm_v[pl.ds(j * 16, 16)] = izeros16
        return 0

    lax.fori_loop(0, BMW // 16, init_bm, 0)

    def do_vreg(buf, voff):
        pk = buf[pl.ds(voff, 16)]
        cell = pk & SENTINEL
        loc = cell - base
        m = (loc >= 0) & (loc < SLICE)
        nact = jnp.sum(m.astype(jnp.int32))

        @pl.when(nact > 0)
        def _():
            locs = jnp.where(m, loc, 0)
            z = plsc.bitcast(buf[pl.ds(CHUNK + voff, 16)], jnp.float32)
            it = plsc.bitcast(buf[pl.ds(2 * CHUNK + voff, 16)], jnp.float32)
            plsc.addupdate_scatter(cnt_v, [locs], ones16, mask=m)
            plsc.addupdate_scatter(sz_v, [locs], z, mask=m)
            plsc.addupdate_scatter(si_v, [locs], it, mask=m)
            hbin = pk >> 20
            sh = hbin + ((locs & 1) << 4)
            bit = jnp.int32(1) << sh
            bmloc = locs >> 1
            oz = plsc.load_gather(mz_v, [locs])
            plsc.store_scatter(mz_v, [locs], jnp.maximum(oz, z), mask=m)
            oi = plsc.load_gather(mi_v, [locs])
            plsc.store_scatter(mi_v, [locs], jnp.maximum(oi, it), mask=m)
            ob = plsc.load_gather(bm_v, [bmloc])
            plsc.store_scatter(bm_v, [bmloc], ob | bit, mask=m)

            @pl.when(nact > 1)
            def _():
                cz = plsc.load_gather(mz_v, [locs])
                ci = plsc.load_gather(mi_v, [locs])
                cb = plsc.load_gather(bm_v, [bmloc])
                pend = m & ((cz < z) | (ci < it) | ((cb & bit) == 0))

                def cond(mv):
                    return jnp.any(mv)

                def body(mv):
                    oz = plsc.load_gather(mz_v, [locs])
                    plsc.store_scatter(mz_v, [locs], jnp.maximum(oz, z), mask=mv)
                    oi = plsc.load_gather(mi_v, [locs])
                    plsc.store_scatter(mi_v, [locs], jnp.maximum(oi, it), mask=mv)
                    ob = plsc.load_gather(bm_v, [bmloc])
                    plsc.store_scatter(bm_v, [bmloc], ob | bit, mask=mv)
                    cz = plsc.load_gather(mz_v, [locs])
                    ci = plsc.load_gather(mi_v, [locs])
                    cb = plsc.load_gather(bm_v, [bmloc])
                    return mv & ((cz < z) | (ci < it) | ((cb & bit) == 0))

                lax.while_loop(cond, body, pend)

    def process(buf):
        def vbody(v, _):
            for u in range(UNROLL):
                do_vreg(buf, (v * UNROLL + u) * 16)
            return 0

        lax.fori_loop(0, CHUNK // 16 // UNROLL, vbody, 0)

    def rec_dma(c, buf, sem):
        return pltpu.make_async_copy(rec_hbm.at[c], buf, sem)

    half = NCHUNK // 2

    def stream_body(k, _):
        rec_dma(2 * k + 1, buf1, sem1).start()
        rec_dma(2 * k, buf0, sem0).wait()
        process(buf0)

        @pl.when(k < half - 1)
        def _():
            rec_dma(2 * k + 2, buf0, sem0).start()

        rec_dma(2 * k + 1, buf1, sem1).wait()
        process(buf1)
        return 0

    rec_dma(0, buf0, sem0).start()
    lax.fori_loop(0, half, stream_body, 0)

    pltpu.sync_copy(cnt_v, cnt_o.at[pl.ds(base, SLICE)])
    pltpu.sync_copy(sz_v, sz_o.at[pl.ds(base, SLICE)])
    pltpu.sync_copy(si_v, si_o.at[pl.ds(base, SLICE)])
    pltpu.sync_copy(mz_v, mz_o.at[pl.ds(base, SLICE)])
    pltpu.sync_copy(mi_v, mi_o.at[pl.ds(base, SLICE)])

    lowmask = jnp.full((16,), 0x3FF, jnp.int32)
    even_idx = lax.iota(jnp.int32, 16) * 2
    pieces = []
    _off = 0
    while _off < BMW // 16:
        _n = min(96, BMW // 16 - _off)
        pieces.append((_off, _n))
        _off += _n
    for piece, (in_voff, n_in) in enumerate(pieces):
        buf = buf0 if piece % 2 == 0 else buf1

        def unpack_vreg(j, _, buf=buf, in_voff=in_voff):
            w = bm_v[pl.ds((in_voff + j) * 16, 16)]
            even = w & lowmask
            odd = (w >> 16) & lowmask
            out_off = j * 32
            plsc.store_scatter(buf, [out_off + even_idx], even)
            plsc.store_scatter(buf, [out_off + even_idx + 1], odd)
            return 0

        lax.fori_loop(0, n_in, unpack_vreg, 0)
        pltpu.sync_copy(
            buf.at[pl.ds(0, n_in * 32)],
            bm_o.at[pl.ds(base + in_voff * 32, n_in * 32)],
        )


_sc_call = functools.partial(
    pl.kernel,
    out_type=[jax.ShapeDtypeStruct((NPAD,), jnp.float32) for _ in range(5)]
    + [jax.ShapeDtypeStruct((NPAD,), jnp.int32)],
    mesh=plsc.VectorSubcoreMesh(core_axis_name="c", subcore_axis_name="s"),
    compiler_params=pltpu.CompilerParams(needs_layout_passes=False),
    scratch_types=[pltpu.VMEM((SLICE,), jnp.float32) for _ in range(5)]
    + [
        pltpu.VMEM((BMW,), jnp.int32),
        pltpu.VMEM((RECW,), jnp.int32),
        pltpu.VMEM((RECW,), jnp.int32),
        pltpu.SemaphoreType.DMA,
        pltpu.SemaphoreType.DMA,
    ],
)(_sc_body)


def _finalize_body(cnt_ref, sz_ref, si_ref, mz_ref, mi_ref, bm_ref, out_ref):
    cnt = cnt_ref[...]
    occ = cnt >= jnp.float32(1.0)
    denom = jnp.where(occ, cnt, jnp.float32(1.0))
    out_ref[0] = jnp.where(occ, mz_ref[...], jnp.float32(0.0))
    out_ref[1] = sz_ref[...] / denom
    out_ref[2] = jnp.where(occ, mi_ref[...], jnp.float32(0.0))
    out_ref[3] = si_ref[...] / denom
    out_ref[4] = jnp.log(jnp.where(occ, cnt + jnp.float32(1.0), jnp.float32(1.0)))
    out_ref[5] = occ.astype(jnp.float32)
    bmi = bm_ref[...]
    for b in range(HB):
        out_ref[6 + b] = ((bmi >> b) & 1).astype(jnp.float32)


_FLAT = NPAD // 128
_FBLK = 152

_finalize_call = pl.pallas_call(
    _finalize_body,
    grid=(_FLAT // _FBLK,),
    in_specs=[
        pl.BlockSpec((_FBLK, 128), lambda i: (i, 0)) for _ in range(6)
    ],
    out_specs=pl.BlockSpec((16, _FBLK, 128), lambda i: (0, i, 0)),
    out_shape=jax.ShapeDtypeStruct((16, _FLAT, 128), jnp.float32),
)


def kernel(points):
    x2 = points[:, 0].reshape(2048, 128)
    y2 = points[:, 1].reshape(2048, 128)
    z2 = points[:, 2].reshape(2048, 128)
    pk2, c02, c12, c22 = _prep_call(x2, y2, z2)
    pk = pk2.reshape(N)
    zbits = lax.bitcast_convert_type(points[:, 2], jnp.int32)
    ibits = lax.bitcast_convert_type(points[:, 3], jnp.int32)
    rec = jnp.stack(
        [pk.reshape(NCHUNK, CHUNK), zbits.reshape(NCHUNK, CHUNK), ibits.reshape(NCHUNK, CHUNK)],
        axis=1,
    ).reshape(NCHUNK, RECW)
    cnt, sz, si, mz, mi, bm = _sc_call(rec)
    grids = [a.reshape(_FLAT, 128) for a in (cnt, sz, si, mz, mi, bm)]
    bev = _finalize_call(*grids).reshape(16, NPAD)[:, :NCELL].reshape(16, NY, NX)
    coors = jnp.stack([c02.reshape(N), c12.reshape(N), c22.reshape(N)], axis=1)
    return coors, bev


# speedup vs baseline: 1.7565x; 1.7565x over previous
import functools

import jax
import jax.numpy as jnp
from jax import lax
from jax.experimental import pallas as pl
from jax.experimental.pallas import tpu as pltpu
from jax.experimental.pallas import tpu_sc as plsc

N = 262144
NX = 848
NY = 848
NCELL = NX * NY
NWORKERS = 32
SLICE = 22496
NPAD = NWORKERS * SLICE
BMW = SLICE // 2
CHUNK = 1024
NCHUNK = N // CHUNK
RECW = 3 * CHUNK
UNROLL = 4
HB = 10
SENTINEL = 0xFFFFF

_F32_NEG_INF = float("-inf")


def _prep_body(x_ref, y_ref, z_ref, pk_ref, c0_ref, c1_ref, c2_ref):
    x = x_ref[...]
    y = y_ref[...]
    z = z_ref[...]
    cx = jnp.floor((x - jnp.float32(-84.8)) / jnp.float32(0.2)).astype(jnp.int32)
    cy = jnp.floor((y - jnp.float32(-84.8)) / jnp.float32(0.2)).astype(jnp.int32)
    cz = jnp.floor((z - jnp.float32(-3.0)) / jnp.float32(5.0)).astype(jnp.int32)
    valid = (
        (cx >= 0) & (cx < NX) & (cy >= 0) & (cy < NY) & (cz >= 0) & (cz < 1)
    )
    cell = cy * NX + cx
    hbin = jnp.clip(
        jnp.floor((z - jnp.float32(-3.0)) / jnp.float32(0.5)).astype(jnp.int32),
        0,
        HB - 1,
    )
    packed = jnp.where(valid, (hbin << 20) | cell, SENTINEL)
    pk_ref[...] = packed
    zero = jnp.zeros_like(cell)
    neg1 = zero - 1
    c0_ref[...] = jnp.where(valid, zero, neg1)
    c1_ref[...] = jnp.where(valid, cy, neg1)
    c2_ref[...] = jnp.where(valid, cx, neg1)


_prep_call = pl.pallas_call(
    _prep_body,
    out_shape=[
        jax.ShapeDtypeStruct((2048, 128), jnp.int32),
        jax.ShapeDtypeStruct((2048, 128), jnp.int32),
        jax.ShapeDtypeStruct((2048, 128), jnp.int32),
        jax.ShapeDtypeStruct((2048, 128), jnp.int32),
    ],
)


def _sc_body(
    rec_hbm,
    cnt_o,
    sz_o,
    si_o,
    mz_o,
    mi_o,
    bm_o,
    cnt_v,
    sz_v,
    si_v,
    mz_v,
    mi_v,
    bm_v,
    buf0,
    buf1,
    sem0,
    sem1,
):
    wid = lax.axis_index("s") * 2 + lax.axis_index("c")
    base = wid * SLICE

    zeros16 = jnp.zeros((16,), jnp.float32)
    ninf16 = jnp.full((16,), _F32_NEG_INF, jnp.float32)
    ones16 = jnp.ones((16,), jnp.float32)
    izeros16 = jnp.zeros((16,), jnp.int32)

    def init(j, _):
        off = j * 16
        cnt_v[pl.ds(off, 16)] = zeros16
        sz_v[pl.ds(off, 16)] = zeros16
        si_v[pl.ds(off, 16)] = zeros16
        mz_v[pl.ds(off, 16)] = ninf16
        mi_v[pl.ds(off, 16)] = ninf16
        return 0

    lax.fori_loop(0, SLICE // 16, init, 0)

    def init_bm(j, _):
        bm_v[pl.ds(j * 16, 16)] = izeros16
        return 0

    lax.fori_loop(0, BMW // 16, init_bm, 0)

    def do_vreg(buf, voff):
        pk = buf[pl.ds(voff, 16)]
        cell = pk & SENTINEL
        loc = cell - base
        m = (loc >= 0) & (loc < SLICE)
        locs = jnp.where(m, loc, 0)
        z = plsc.bitcast(buf[pl.ds(CHUNK + voff, 16)], jnp.float32)
        it = plsc.bitcast(buf[pl.ds(2 * CHUNK + voff, 16)], jnp.float32)
        plsc.addupdate_scatter(cnt_v, [locs], ones16, mask=m)
        plsc.addupdate_scatter(sz_v, [locs], z, mask=m)
        plsc.addupdate_scatter(si_v, [locs], it, mask=m)
        hbin = pk >> 20
        sh = hbin + ((locs & 1) << 4)
        bit = jnp.int32(1) << sh
        bmloc = locs >> 1
        oz = plsc.load_gather(mz_v, [locs])
        plsc.store_scatter(mz_v, [locs], jnp.maximum(oz, z), mask=m)
        oi = plsc.load_gather(mi_v, [locs])
        plsc.store_scatter(mi_v, [locs], jnp.maximum(oi, it), mask=m)
        ob = plsc.load_gather(bm_v, [bmloc])
        plsc.store_scatter(bm_v, [bmloc], ob | bit, mask=m)
        cz = plsc.load_gather(mz_v, [locs])
        ci = plsc.load_gather(bm_v, [bmloc])
        czf = plsc.load_gather(mi_v, [locs])
        pend = m & ((cz < z) | (czf < it) | ((ci & bit) == 0))
        return pend, locs, bmloc, z, it, bit

    def retry_vreg(state):
        pend, locs, bmloc, z, it, bit = state

        def cond(mv):
            return jnp.any(mv)

        def body(mv):
            oz = plsc.load_gather(mz_v, [locs])
            plsc.store_scatter(mz_v, [locs], jnp.maximum(oz, z), mask=mv)
            oi = plsc.load_gather(mi_v, [locs])
            plsc.store_scatter(mi_v, [locs], jnp.maximum(oi, it), mask=mv)
            ob = plsc.load_gather(bm_v, [bmloc])
            plsc.store_scatter(bm_v, [bmloc], ob | bit, mask=mv)
            cz = plsc.load_gather(mz_v, [locs])
            ci = plsc.load_gather(mi_v, [locs])
            cb = plsc.load_gather(bm_v, [bmloc])
            return mv & ((cz < z) | (ci < it) | ((cb & bit) == 0))

        lax.while_loop(cond, body, pend)

    def process(buf):
        def vbody(v, _):
            states = [
                do_vreg(buf, (v * UNROLL + u) * 16) for u in range(UNROLL)
            ]
            anyp = states[0][0]
            for s in states[1:]:
                anyp = anyp | s[0]

            @pl.when(jnp.any(anyp))
            def _():
                for s in states:
                    retry_vreg(s)

            return 0

        lax.fori_loop(0, CHUNK // 16 // UNROLL, vbody, 0)

    def rec_dma(c, buf, sem):
        return pltpu.make_async_copy(rec_hbm.at[c], buf, sem)

    half = NCHUNK // 2

    def stream_body(k, _):
        rec_dma(2 * k + 1, buf1, sem1).start()
        rec_dma(2 * k, buf0, sem0).wait()
        process(buf0)

        @pl.when(k < half - 1)
        def _():
            rec_dma(2 * k + 2, buf0, sem0).start()

        rec_dma(2 * k + 1, buf1, sem1).wait()
        process(buf1)
        return 0

    rec_dma(0, buf0, sem0).start()
    lax.fori_loop(0, half, stream_body, 0)

    pltpu.sync_copy(cnt_v, cnt_o.at[pl.ds(base, SLICE)])
    pltpu.sync_copy(sz_v, sz_o.at[pl.ds(base, SLICE)])
    pltpu.sync_copy(si_v, si_o.at[pl.ds(base, SLICE)])
    pltpu.sync_copy(mz_v, mz_o.at[pl.ds(base, SLICE)])
    pltpu.sync_copy(mi_v, mi_o.at[pl.ds(base, SLICE)])

    lowmask = jnp.full((16,), 0x3FF, jnp.int32)
    even_idx = lax.iota(jnp.int32, 16) * 2
    pieces = []
    _off = 0
    while _off < BMW // 16:
        _n = min(96, BMW // 16 - _off)
        pieces.append((_off, _n))
        _off += _n
    for piece, (in_voff, n_in) in enumerate(pieces):
        buf = buf0 if piece % 2 == 0 else buf1

        def unpack_vreg(j, _, buf=buf, in_voff=in_voff):
            w = bm_v[pl.ds((in_voff + j) * 16, 16)]
            even = w & lowmask
            odd = (w >> 16) & lowmask
            out_off = j * 32
            plsc.store_scatter(buf, [out_off + even_idx], even)
            plsc.store_scatter(buf, [out_off + even_idx + 1], odd)
            return 0

        lax.fori_loop(0, n_in, unpack_vreg, 0)
        pltpu.sync_copy(
            buf.at[pl.ds(0, n_in * 32)],
            bm_o.at[pl.ds(base + in_voff * 32, n_in * 32)],
        )


_sc_call = functools.partial(
    pl.kernel,
    out_type=[jax.ShapeDtypeStruct((NPAD,), jnp.float32) for _ in range(5)]
    + [jax.ShapeDtypeStruct((NPAD,), jnp.int32)],
    mesh=plsc.VectorSubcoreMesh(core_axis_name="c", subcore_axis_name="s"),
    compiler_params=pltpu.CompilerParams(needs_layout_passes=False),
    scratch_types=[pltpu.VMEM((SLICE,), jnp.float32) for _ in range(5)]
    + [
        pltpu.VMEM((BMW,), jnp.int32),
        pltpu.VMEM((RECW,), jnp.int32),
        pltpu.VMEM((RECW,), jnp.int32),
        pltpu.SemaphoreType.DMA,
        pltpu.SemaphoreType.DMA,
    ],
)(_sc_body)


def _finalize_body(cnt_ref, sz_ref, si_ref, mz_ref, mi_ref, bm_ref, out_ref):
    cnt = cnt_ref[...]
    occ = cnt >= jnp.float32(1.0)
    denom = jnp.where(occ, cnt, jnp.float32(1.0))
    out_ref[0] = jnp.where(occ, mz_ref[...], jnp.float32(0.0))
    out_ref[1] = sz_ref[...] / denom
    out_ref[2] = jnp.where(occ, mi_ref[...], jnp.float32(0.0))
    out_ref[3] = si_ref[...] / denom
    out_ref[4] = jnp.log(jnp.where(occ, cnt + jnp.float32(1.0), jnp.float32(1.0)))
    out_ref[5] = occ.astype(jnp.float32)
    bmi = bm_ref[...]
    for b in range(HB):
        out_ref[6 + b] = ((bmi >> b) & 1).astype(jnp.float32)


_FLAT = NPAD // 128
_FBLK = 152

_finalize_call = pl.pallas_call(
    _finalize_body,
    grid=(_FLAT // _FBLK,),
    in_specs=[
        pl.BlockSpec((_FBLK, 128), lambda i: (i, 0)) for _ in range(6)
    ],
    out_specs=pl.BlockSpec((16, _FBLK, 128), lambda i: (0, i, 0)),
    out_shape=jax.ShapeDtypeStruct((16, _FLAT, 128), jnp.float32),
)


def kernel(points):
    x2 = points[:, 0].reshape(2048, 128)
    y2 = points[:, 1].reshape(2048, 128)
    z2 = points[:, 2].reshape(2048, 128)
    pk2, c02, c12, c22 = _prep_call(x2, y2, z2)
    pk = pk2.reshape(N)
    zbits = lax.bitcast_convert_type(points[:, 2], jnp.int32)
    ibits = lax.bitcast_convert_type(points[:, 3], jnp.int32)
    rec = jnp.stack(
        [pk.reshape(NCHUNK, CHUNK), zbits.reshape(NCHUNK, CHUNK), ibits.reshape(NCHUNK, CHUNK)],
        axis=1,
    ).reshape(NCHUNK, RECW)
    cnt, sz, si, mz, mi, bm = _sc_call(rec)
    grids = [a.reshape(_FLAT, 128) for a in (cnt, sz, si, mz, mi, bm)]
    bev = _finalize_call(*grids).reshape(16, NPAD)[:, :NCELL].reshape(16, NY, NX)
    coors = jnp.stack([c02.reshape(N), c12.reshape(N), c22.reshape(N)], axis=1)
    return coors, bev


# speedup vs baseline: 1.9123x; 1.0887x over previous
import functools

import jax
import jax.numpy as jnp
from jax import lax
from jax.experimental import pallas as pl
from jax.experimental.pallas import tpu as pltpu
from jax.experimental.pallas import tpu_sc as plsc

N = 262144
NX = 848
NY = 848
NCELL = NX * NY
NWORKERS = 32
SLICE = 22496
NPAD = NWORKERS * SLICE
BMW = SLICE // 2
CHUNK = 1024
NCHUNK = N // CHUNK
RECW = 3 * CHUNK
UNROLL = 8
HB = 10
SENTINEL = 0xFFFFF

_F32_NEG_INF = float("-inf")


def _prep_body(x_ref, y_ref, z_ref, pk_ref, c0_ref, c1_ref, c2_ref):
    x = x_ref[...]
    y = y_ref[...]
    z = z_ref[...]
    cx = jnp.floor((x - jnp.float32(-84.8)) / jnp.float32(0.2)).astype(jnp.int32)
    cy = jnp.floor((y - jnp.float32(-84.8)) / jnp.float32(0.2)).astype(jnp.int32)
    cz = jnp.floor((z - jnp.float32(-3.0)) / jnp.float32(5.0)).astype(jnp.int32)
    valid = (
        (cx >= 0) & (cx < NX) & (cy >= 0) & (cy < NY) & (cz >= 0) & (cz < 1)
    )
    cell = cy * NX + cx
    hbin = jnp.clip(
        jnp.floor((z - jnp.float32(-3.0)) / jnp.float32(0.5)).astype(jnp.int32),
        0,
        HB - 1,
    )
    packed = jnp.where(valid, (hbin << 20) | cell, SENTINEL)
    pk_ref[...] = packed
    zero = jnp.zeros_like(cell)
    neg1 = zero - 1
    c0_ref[...] = jnp.where(valid, zero, neg1)
    c1_ref[...] = jnp.where(valid, cy, neg1)
    c2_ref[...] = jnp.where(valid, cx, neg1)


_prep_call = pl.pallas_call(
    _prep_body,
    out_shape=[
        jax.ShapeDtypeStruct((2048, 128), jnp.int32),
        jax.ShapeDtypeStruct((2048, 128), jnp.int32),
        jax.ShapeDtypeStruct((2048, 128), jnp.int32),
        jax.ShapeDtypeStruct((2048, 128), jnp.int32),
    ],
)


def _sc_body(
    rec_hbm,
    cnt_o,
    sz_o,
    si_o,
    mz_o,
    mi_o,
    bm_o,
    cnt_v,
    sz_v,
    si_v,
    mz_v,
    mi_v,
    bm_v,
    buf0,
    buf1,
    sem0,
    sem1,
):
    wid = lax.axis_index("s") * 2 + lax.axis_index("c")
    base = wid * SLICE

    zeros16 = jnp.zeros((16,), jnp.float32)
    ninf16 = jnp.full((16,), _F32_NEG_INF, jnp.float32)
    ones16 = jnp.ones((16,), jnp.float32)
    izeros16 = jnp.zeros((16,), jnp.int32)

    def init(j, _):
        off = j * 16
        cnt_v[pl.ds(off, 16)] = zeros16
        sz_v[pl.ds(off, 16)] = zeros16
        si_v[pl.ds(off, 16)] = zeros16
        mz_v[pl.ds(off, 16)] = ninf16
        mi_v[pl.ds(off, 16)] = ninf16
        return 0

    lax.fori_loop(0, SLICE // 16, init, 0)

    def init_bm(j, _):
        bm_v[pl.ds(j * 16, 16)] = izeros16
        return 0

    lax.fori_loop(0, BMW // 16, init_bm, 0)

    def do_vreg(buf, voff):
        pk = buf[pl.ds(voff, 16)]
        cell = pk & SENTINEL
        loc = cell - base
        m = (loc >= 0) & (loc < SLICE)
        locs = jnp.where(m, loc, 0)
        z = plsc.bitcast(buf[pl.ds(CHUNK + voff, 16)], jnp.float32)
        it = plsc.bitcast(buf[pl.ds(2 * CHUNK + voff, 16)], jnp.float32)
        plsc.addupdate_scatter(cnt_v, [locs], ones16, mask=m)
        plsc.addupdate_scatter(sz_v, [locs], z, mask=m)
        plsc.addupdate_scatter(si_v, [locs], it, mask=m)
        hbin = pk >> 20
        sh = hbin + ((locs & 1) << 4)
        bit = jnp.int32(1) << sh
        bmloc = locs >> 1
        oz = plsc.load_gather(mz_v, [locs])
        plsc.store_scatter(mz_v, [locs], jnp.maximum(oz, z), mask=m)
        oi = plsc.load_gather(mi_v, [locs])
        plsc.store_scatter(mi_v, [locs], jnp.maximum(oi, it), mask=m)
        ob = plsc.load_gather(bm_v, [bmloc])
        plsc.store_scatter(bm_v, [bmloc], ob | bit, mask=m)
        cz = plsc.load_gather(mz_v, [locs])
        ci = plsc.load_gather(bm_v, [bmloc])
        czf = plsc.load_gather(mi_v, [locs])
        pend = m & ((cz < z) | (czf < it) | ((ci & bit) == 0))
        return pend, locs, bmloc, z, it, bit

    def retry_vreg(state):
        pend, locs, bmloc, z, it, bit = state

        def cond(mv):
            return jnp.any(mv)

        def body(mv):
            oz = plsc.load_gather(mz_v, [locs])
            plsc.store_scatter(mz_v, [locs], jnp.maximum(oz, z), mask=mv)
            oi = plsc.load_gather(mi_v, [locs])
            plsc.store_scatter(mi_v, [locs], jnp.maximum(oi, it), mask=mv)
            ob = plsc.load_gather(bm_v, [bmloc])
            plsc.store_scatter(bm_v, [bmloc], ob | bit, mask=mv)
            cz = plsc.load_gather(mz_v, [locs])
            ci = plsc.load_gather(mi_v, [locs])
            cb = plsc.load_gather(bm_v, [bmloc])
            return mv & ((cz < z) | (ci < it) | ((cb & bit) == 0))

        lax.while_loop(cond, body, pend)

    def process(buf):
        def vbody(v, _):
            states = [
                do_vreg(buf, (v * UNROLL + u) * 16) for u in range(UNROLL)
            ]
            anyp = states[0][0]
            for s in states[1:]:
                anyp = anyp | s[0]

            @pl.when(jnp.any(anyp))
            def _():
                for s in states:
                    retry_vreg(s)

            return 0

        lax.fori_loop(0, CHUNK // 16 // UNROLL, vbody, 0)

    def rec_dma(c, buf, sem):
        return pltpu.make_async_copy(rec_hbm.at[c], buf, sem)

    half = NCHUNK // 2

    def stream_body(k, _):
        rec_dma(2 * k + 1, buf1, sem1).start()
        rec_dma(2 * k, buf0, sem0).wait()
        process(buf0)

        @pl.when(k < half - 1)
        def _():
            rec_dma(2 * k + 2, buf0, sem0).start()

        rec_dma(2 * k + 1, buf1, sem1).wait()
        process(buf1)
        return 0

    rec_dma(0, buf0, sem0).start()
    lax.fori_loop(0, half, stream_body, 0)

    pltpu.sync_copy(cnt_v, cnt_o.at[pl.ds(base, SLICE)])
    pltpu.sync_copy(sz_v, sz_o.at[pl.ds(base, SLICE)])
    pltpu.sync_copy(si_v, si_o.at[pl.ds(base, SLICE)])
    pltpu.sync_copy(mz_v, mz_o.at[pl.ds(base, SLICE)])
    pltpu.sync_copy(mi_v, mi_o.at[pl.ds(base, SLICE)])

    lowmask = jnp.full((16,), 0x3FF, jnp.int32)
    even_idx = lax.iota(jnp.int32, 16) * 2
    pieces = []
    _off = 0
    while _off < BMW // 16:
        _n = min(96, BMW // 16 - _off)
        pieces.append((_off, _n))
        _off += _n
    for piece, (in_voff, n_in) in enumerate(pieces):
        buf = buf0 if piece % 2 == 0 else buf1

        def unpack_vreg(j, _, buf=buf, in_voff=in_voff):
            w = bm_v[pl.ds((in_voff + j) * 16, 16)]
            even = w & lowmask
            odd = (w >> 16) & lowmask
            out_off = j * 32
            plsc.store_scatter(buf, [out_off + even_idx], even)
            plsc.store_scatter(buf, [out_off + even_idx + 1], odd)
            return 0

        lax.fori_loop(0, n_in, unpack_vreg, 0)
        pltpu.sync_copy(
            buf.at[pl.ds(0, n_in * 32)],
            bm_o.at[pl.ds(base + in_voff * 32, n_in * 32)],
        )


_sc_call = functools.partial(
    pl.kernel,
    out_type=[jax.ShapeDtypeStruct((NPAD,), jnp.float32) for _ in range(5)]
    + [jax.ShapeDtypeStruct((NPAD,), jnp.int32)],
    mesh=plsc.VectorSubcoreMesh(core_axis_name="c", subcore_axis_name="s"),
    compiler_params=pltpu.CompilerParams(needs_layout_passes=False),
    scratch_types=[pltpu.VMEM((SLICE,), jnp.float32) for _ in range(5)]
    + [
        pltpu.VMEM((BMW,), jnp.int32),
        pltpu.VMEM((RECW,), jnp.int32),
        pltpu.VMEM((RECW,), jnp.int32),
        pltpu.SemaphoreType.DMA,
        pltpu.SemaphoreType.DMA,
    ],
)(_sc_body)


def _finalize_body(cnt_ref, sz_ref, si_ref, mz_ref, mi_ref, bm_ref, out_ref):
    cnt = cnt_ref[...]
    occ = cnt >= jnp.float32(1.0)
    denom = jnp.where(occ, cnt, jnp.float32(1.0))
    out_ref[0] = jnp.where(occ, mz_ref[...], jnp.float32(0.0))
    out_ref[1] = sz_ref[...] / denom
    out_ref[2] = jnp.where(occ, mi_ref[...], jnp.float32(0.0))
    out_ref[3] = si_ref[...] / denom
    out_ref[4] = jnp.log(jnp.where(occ, cnt + jnp.float32(1.0), jnp.float32(1.0)))
    out_ref[5] = occ.astype(jnp.float32)
    bmi = bm_ref[...]
    for b in range(HB):
        out_ref[6 + b] = ((bmi >> b) & 1).astype(jnp.float32)


_FLAT = NPAD // 128
_FBLK = 152

_finalize_call = pl.pallas_call(
    _finalize_body,
    grid=(_FLAT // _FBLK,),
    in_specs=[
        pl.BlockSpec((_FBLK, 128), lambda i: (i, 0)) for _ in range(6)
    ],
    out_specs=pl.BlockSpec((16, _FBLK, 128), lambda i: (0, i, 0)),
    out_shape=jax.ShapeDtypeStruct((16, _FLAT, 128), jnp.float32),
)


def kernel(points):
    x2 = points[:, 0].reshape(2048, 128)
    y2 = points[:, 1].reshape(2048, 128)
    z2 = points[:, 2].reshape(2048, 128)
    pk2, c02, c12, c22 = _prep_call(x2, y2, z2)
    pk = pk2.reshape(N)
    zbits = lax.bitcast_convert_type(points[:, 2], jnp.int32)
    ibits = lax.bitcast_convert_type(points[:, 3], jnp.int32)
    rec = jnp.stack(
        [pk.reshape(NCHUNK, CHUNK), zbits.reshape(NCHUNK, CHUNK), ibits.reshape(NCHUNK, CHUNK)],
        axis=1,
    ).reshape(NCHUNK, RECW)
    cnt, sz, si, mz, mi, bm = _sc_call(rec)
    grids = [a.reshape(_FLAT, 128) for a in (cnt, sz, si, mz, mi, bm)]
    bev = _finalize_call(*grids).reshape(16, NPAD)[:, :NCELL].reshape(16, NY, NX)
    coors = jnp.stack([c02.reshape(N), c12.reshape(N), c22.reshape(N)], axis=1)
    return coors, bev
